# Initial kernel scaffold; baseline (speedup 1.0000x reference)
import jax
import jax.numpy as jnp
from jax.experimental import pallas as pl


def kernel(x, edge_index, batch, W1, b1, W2, b2, Wf, bf):
    raise NotImplementedError("write your pallas kernel here")


# speedup vs baseline: 29.8941x; 29.8941x over previous
import functools

import jax
import jax.numpy as jnp
from jax import lax
from jax.experimental import pallas as pl
from jax.experimental.pallas import tpu as pltpu
from jax.experimental.pallas import tpu_sc as plsc

N = 100000
E = 1600000
G = 128

NPAD = 100352
EPAD = 1605632
ROWS = EPAD // 128
SL = 128
GSL = 4
GROUP_E = GSL * SL
STRIPE = NPAD // 16
BLK = 1024

_mesh = plsc.VectorSubcoreMesh(core_axis_name="c", subcore_axis_name="s")


def _make_agg_kernel(slices_per_tile, split_by_core):
    n_iter = slices_per_tile // (2 * GSL)

    @functools.partial(
        pl.kernel,
        out_type=jax.ShapeDtypeStruct((2, NPAD, 16), jnp.float32),
        mesh=_mesh,
        scratch_types=[
            pltpu.VMEM((2, GSL, SL), jnp.int32),
            pltpu.VMEM((2, GSL, SL), jnp.int32),
            pltpu.VMEM((2, GROUP_E, 16), jnp.float32),
            pltpu.VMEM_SHARED((NPAD, 16), jnp.float32),
            pltpu.SemaphoreType.DMA,
            pltpu.SemaphoreType.DMA,
            pltpu.SemaphoreType.DMA,
            pltpu.SemaphoreType.DMA,
        ],
        compiler_params=pltpu.CompilerParams(use_tc_tiling_on_sc=False),
    )
    def agg_kernel(sA, sB, dI, tab, zeros2, out, sidx, didx, rows, acc, gs0, gs1, ss0, ss1):
        c = lax.axis_index("c")
        s = lax.axis_index("s")
        stripe0 = s * STRIPE
        if split_by_core:
            base = (c * 16 + s) * slices_per_tile
        else:
            base = s * slices_per_tile

        pltpu.sync_copy(zeros2.at[pl.ds(stripe0, STRIPE)], acc.at[pl.ds(stripe0, STRIPE)])
        plsc.subcore_barrier()

        def drain_group(sem):
            pltpu.make_async_copy(
                tab.at[pl.ds(0, GROUP_E)], rows.at[0, pl.ds(0, GROUP_E)], sem
            ).wait()

        def run_group(gidx, b, gsem, ssem):
            sb = base + gidx * GSL

            @pl.when(c == 0)
            def _():
                pltpu.sync_copy(sA.at[pl.ds(sb, GSL)], sidx.at[b])

            @pl.when(c == 1)
            def _():
                pltpu.sync_copy(sB.at[pl.ds(sb, GSL)], sidx.at[b])

            pltpu.sync_copy(dI.at[pl.ds(sb, GSL)], didx.at[b])
            gd = []
            for j in range(GSL):
                gd.append(pltpu.async_copy(
                    tab.at[sidx.at[b, j]], rows.at[b, pl.ds(j * SL, SL)], gsem))
            for d in gd:
                d.wait()
            for j in range(GSL):
                pltpu.async_copy(
                    rows.at[b, pl.ds(j * SL, SL)], acc.at[didx.at[b, j]], ssem,
                    add=True)

        def body(t, carry):
            @pl.when(t > 0)
            def _():
                drain_group(ss0)

            run_group(2 * t, 0, gs0, ss0)

            @pl.when(t > 0)
            def _():
                drain_group(ss1)

            run_group(2 * t + 1, 1, gs1, ss1)
            return carry

        lax.fori_loop(0, n_iter, body, 0)
        drain_group(ss0)
        drain_group(ss1)
        plsc.subcore_barrier()
        pltpu.sync_copy(acc.at[pl.ds(stripe0, STRIPE)], out.at[c, pl.ds(stripe0, STRIPE)])

    return agg_kernel


_agg_l1 = _make_agg_kernel(EPAD // 128 // 32, split_by_core=True)
_agg_l2 = _make_agg_kernel(EPAD // 128 // 16, split_by_core=False)


DEG_GSL = 49


@functools.partial(
    pl.kernel,
    out_type=jax.ShapeDtypeStruct((2 * NPAD,), jnp.float32),
    mesh=_mesh,
    scratch_types=[
        pltpu.VMEM((DEG_GSL, SL), jnp.int32),
        pltpu.VMEM((SL,), jnp.float32),
        pltpu.VMEM_SHARED((NPAD,), jnp.float32),
        pltpu.SemaphoreType.DMA,
        pltpu.SemaphoreType.DMA,
    ],
    compiler_params=pltpu.CompilerParams(use_tc_tiling_on_sc=False),
)
def _deg_kernel(dI, zeros1, out, didx, ones, deg, sem, ssem):
    c = lax.axis_index("c")
    s = lax.axis_index("s")
    w = c * 16 + s
    for i in range(SL // 16):
        ones[pl.ds(i * 16, 16)] = jnp.ones((16,), jnp.float32)
    stripe0 = s * STRIPE
    pltpu.sync_copy(zeros1.at[pl.ds(stripe0, STRIPE)], deg.at[pl.ds(stripe0, STRIPE)])
    plsc.subcore_barrier()

    base = w * (ROWS // 32)

    def body(t, carry):
        pltpu.sync_copy(dI.at[pl.ds(base + t * DEG_GSL, DEG_GSL)], didx)
        descs = []
        for j in range(DEG_GSL):
            descs.append(pltpu.async_copy(ones, deg.at[didx.at[j]], ssem, add=True))
        for d in descs:
            d.wait()
        return carry

    lax.fori_loop(0, ROWS // 32 // DEG_GSL, body, 0)
    plsc.subcore_barrier()
    pltpu.sync_copy(deg.at[pl.ds(stripe0, STRIPE)],
                    out.at[pl.ds(c * NPAD + stripe0, STRIPE)])


def _k1_body(deg_ref, x_ref, dis_ref, q_ref):
    d = deg_ref[0, :] + deg_ref[1, :] + 1.0
    r = lax.rsqrt(d)
    dis_ref[...] = r[:, None]
    q_ref[...] = x_ref[...] * r[:, None]


def _k3_body(agg_ref, q_ref, dis_ref, W1_ref, b1_ref, W2_ref, p2_ref, hw2_ref):
    a = agg_ref[0] + agg_ref[1] + q_ref[...]
    t1 = dis_ref[...] * a
    h1 = jnp.dot(t1, W1_ref[...], preferred_element_type=jnp.float32) + b1_ref[...]
    h1 = jnp.maximum(h1, 0.0)
    hw2 = jnp.dot(h1, W2_ref[...], preferred_element_type=jnp.float32)
    hw2_ref[...] = hw2
    p2 = dis_ref[...] * hw2
    p2_ref[0] = p2[:, :16]
    p2_ref[1] = p2[:, 16:]


def _k5_body(agg_ref, hw2_ref, dis_ref, batch_ref, b2_ref, Wf_ref, bf_ref,
             out_ref, acc_ref):
    i = pl.program_id(0)

    @pl.when(i == 0)
    def _():
        acc_ref[...] = jnp.zeros_like(acc_ref)

    aggcat = jnp.concatenate([agg_ref[0], agg_ref[1]], axis=1)
    z = dis_ref[...] * (aggcat + dis_ref[...] * hw2_ref[...]) + b2_ref[...]
    h2 = jnp.maximum(z, 0.0)
    b = batch_ref[...]
    oh = (b[:, None] == lax.broadcasted_iota(jnp.int32, (BLK, G), 1)).astype(jnp.float32)
    acc_ref[...] += lax.dot_general(
        oh, h2, (((0,), (0,)), ((), ())), preferred_element_type=jnp.float32)

    @pl.when(i == pl.num_programs(0) - 1)
    def _():
        gg = acc_ref[...]
        out_ref[...] = jax.nn.sigmoid(
            jnp.dot(gg, Wf_ref[...], preferred_element_type=jnp.float32)
            + bf_ref[...])


def kernel(x, edge_index, batch, W1, b1, W2, b2, Wf, bf):
    src = edge_index[0]
    dst = edge_index[1]
    srcp = jnp.concatenate([src, jnp.zeros((EPAD - E,), jnp.int32)])
    dstp = jnp.concatenate([dst, jnp.full((EPAD - E,), N, jnp.int32)])
    src3 = srcp.reshape(ROWS, SL)
    srcB3 = (srcp + NPAD).reshape(ROWS, SL)
    dst3 = dstp.reshape(ROWS, SL)
    xp = jnp.pad(x, ((0, NPAD - N), (0, 6)))
    batchp = jnp.pad(batch, (0, NPAD - N), constant_values=G)
    W1p = jnp.pad(W1, ((0, 6), (0, 0)))
    zeros1 = jnp.zeros((NPAD,), jnp.float32)
    zeros2 = jnp.zeros((NPAD, 16), jnp.float32)

    deg = _deg_kernel(dst3, zeros1).reshape(2, NPAD)

    nblk = NPAD // BLK
    dis, q = pl.pallas_call(
        _k1_body,
        grid=(nblk,),
        in_specs=[
            pl.BlockSpec((2, BLK), lambda i: (0, i)),
            pl.BlockSpec((BLK, 16), lambda i: (i, 0)),
        ],
        out_specs=[
            pl.BlockSpec((BLK, 1), lambda i: (i, 0)),
            pl.BlockSpec((BLK, 16), lambda i: (i, 0)),
        ],
        out_shape=[
            jax.ShapeDtypeStruct((NPAD, 1), jnp.float32),
            jax.ShapeDtypeStruct((NPAD, 16), jnp.float32),
        ],
    )(deg, xp)

    agg1 = _agg_l1(src3, src3, dst3, q, zeros2)

    p2, hw2 = pl.pallas_call(
        _k3_body,
        grid=(nblk,),
        in_specs=[
            pl.BlockSpec((2, BLK, 16), lambda i: (0, i, 0)),
            pl.BlockSpec((BLK, 16), lambda i: (i, 0)),
            pl.BlockSpec((BLK, 1), lambda i: (i, 0)),
            pl.BlockSpec((16, 64), lambda i: (0, 0)),
            pl.BlockSpec((1, 64), lambda i: (0, 0)),
            pl.BlockSpec((64, 32), lambda i: (0, 0)),
        ],
        out_specs=[
            pl.BlockSpec((2, BLK, 16), lambda i: (0, i, 0)),
            pl.BlockSpec((BLK, 32), lambda i: (i, 0)),
        ],
        out_shape=[
            jax.ShapeDtypeStruct((2, NPAD, 16), jnp.float32),
            jax.ShapeDtypeStruct((NPAD, 32), jnp.float32),
        ],
    )(agg1, q, dis, W1p, b1.reshape(1, 64), W2)

    p2flat = p2.reshape(2 * NPAD, 16)
    agg2 = _agg_l2(src3, srcB3, dst3, p2flat, zeros2)

    out = pl.pallas_call(
        _k5_body,
        grid=(nblk,),
        in_specs=[
            pl.BlockSpec((2, BLK, 16), lambda i: (0, i, 0)),
            pl.BlockSpec((BLK, 32), lambda i: (i, 0)),
            pl.BlockSpec((BLK, 1), lambda i: (i, 0)),
            pl.BlockSpec((BLK,), lambda i: (i,)),
            pl.BlockSpec((1, 32), lambda i: (0, 0)),
            pl.BlockSpec((32, 1), lambda i: (0, 0)),
            pl.BlockSpec((1, 1), lambda i: (0, 0)),
        ],
        out_specs=pl.BlockSpec((G, 1), lambda i: (0, 0)),
        out_shape=jax.ShapeDtypeStruct((G, 1), jnp.float32),
        scratch_shapes=[pltpu.VMEM((G, 32), jnp.float32)],
    )(agg2, hw2, dis, batchp, b2.reshape(1, 32), Wf, bf.reshape(1, 1))

    return out


# speedup vs baseline: 32.3632x; 1.0826x over previous
import functools

import jax
import jax.numpy as jnp
from jax import lax
from jax.experimental import pallas as pl
from jax.experimental.pallas import tpu as pltpu
from jax.experimental.pallas import tpu_sc as plsc

N = 100000
E = 1600000
G = 128

NPAD = 100352
EPAD = 1605632
GE = 784
NGROUPS = EPAD // GE
STRIPE = NPAD // 16
BLK = NPAD // 32

_mesh = plsc.VectorSubcoreMesh(core_axis_name="c", subcore_axis_name="s")
_sc_params = pltpu.CompilerParams(use_tc_tiling_on_sc=False)


def _make_agg_kernel(groups_per_tile, split_by_core):
    n_iter = groups_per_tile // 2

    @functools.partial(
        pl.kernel,
        out_type=jax.ShapeDtypeStruct((2, NPAD, 16), jnp.float32),
        mesh=_mesh,
        scratch_types=[
            pltpu.VMEM((2, GE), jnp.int32),
            pltpu.VMEM((2, GE), jnp.int32),
            pltpu.VMEM((2, GE, 16), jnp.float32),
            pltpu.VMEM_SHARED((NPAD, 16), jnp.float32),
            pltpu.SemaphoreType.DMA,
            pltpu.SemaphoreType.DMA,
            pltpu.SemaphoreType.DMA,
            pltpu.SemaphoreType.DMA,
        ],
        compiler_params=_sc_params,
    )
    def agg_kernel(sA, sB, dI, tab, out, sidx, didx, rows, acc, gs0, gs1, ss0, ss1):
        c = lax.axis_index("c")
        s = lax.axis_index("s")
        stripe0 = s * STRIPE
        if split_by_core:
            gbase = (c * 16 + s) * groups_per_tile
        else:
            gbase = s * groups_per_tile

        def zstore(r, carry):
            rows[0, r, :] = jnp.zeros((16,), jnp.float32)
            return carry

        lax.fori_loop(0, GE, zstore, 0)
        zd = []
        for k in range(STRIPE // GE):
            zd.append(pltpu.async_copy(
                rows.at[0], acc.at[pl.ds(stripe0 + k * GE, GE)], gs0))
        for d in zd:
            d.wait()
        plsc.subcore_barrier()

        def drain_group(sem):
            pltpu.make_async_copy(tab.at[pl.ds(0, GE)], rows.at[0], sem).wait()

        def run_group(gidx, b, gsem, ssem):
            eb = (gbase + gidx) * GE

            @pl.when(c == 0)
            def _():
                pltpu.sync_copy(sA.at[pl.ds(eb, GE)], sidx.at[b])

            @pl.when(c == 1)
            def _():
                pltpu.sync_copy(sB.at[pl.ds(eb, GE)], sidx.at[b])

            pltpu.sync_copy(dI.at[pl.ds(eb, GE)], didx.at[b])
            pltpu.async_copy(tab.at[sidx.at[b]], rows.at[b], gsem).wait()
            pltpu.async_copy(rows.at[b], acc.at[didx.at[b]], ssem, add=True)

        def body(t, carry):
            @pl.when(t > 0)
            def _():
                drain_group(ss0)

            run_group(2 * t, 0, gs0, ss0)

            @pl.when(t > 0)
            def _():
                drain_group(ss1)

            run_group(2 * t + 1, 1, gs1, ss1)
            return carry

        lax.fori_loop(0, n_iter, body, 0)
        drain_group(ss0)
        drain_group(ss1)
        plsc.subcore_barrier()
        pltpu.sync_copy(acc.at[pl.ds(stripe0, STRIPE)], out.at[c, pl.ds(stripe0, STRIPE)])

    return agg_kernel


_agg_l1 = _make_agg_kernel(NGROUPS // 32, split_by_core=True)
_agg_l2 = _make_agg_kernel(NGROUPS // 16, split_by_core=False)

DEG_GE = 6272


@functools.partial(
    pl.kernel,
    out_type=jax.ShapeDtypeStruct((2 * NPAD,), jnp.float32),
    mesh=_mesh,
    scratch_types=[
        pltpu.VMEM((DEG_GE,), jnp.int32),
        pltpu.VMEM((DEG_GE,), jnp.float32),
        pltpu.VMEM_SHARED((NPAD,), jnp.float32),
        pltpu.SemaphoreType.DMA,
        pltpu.SemaphoreType.DMA,
    ],
    compiler_params=_sc_params,
)
def _deg_kernel(dI, out, didx, ones, deg, sem, ssem):
    c = lax.axis_index("c")
    s = lax.axis_index("s")
    w = c * 16 + s

    def zstore(r, carry):
        ones[pl.ds(r * 16, 16)] = jnp.zeros((16,), jnp.float32)
        return carry

    lax.fori_loop(0, DEG_GE // 16, zstore, 0)
    stripe0 = s * STRIPE
    pltpu.sync_copy(ones.at[pl.ds(0, STRIPE)], deg.at[pl.ds(stripe0, STRIPE)])

    def ostore(r, carry):
        ones[pl.ds(r * 16, 16)] = jnp.ones((16,), jnp.float32)
        return carry

    lax.fori_loop(0, DEG_GE // 16, ostore, 0)
    plsc.subcore_barrier()

    ebase = w * (EPAD // 32)

    def body(t, carry):
        pltpu.sync_copy(dI.at[pl.ds(ebase + t * DEG_GE, DEG_GE)], didx)
        pltpu.async_copy(ones, deg.at[didx], ssem, add=True).wait()
        return carry

    lax.fori_loop(0, EPAD // 32 // DEG_GE, body, 0)
    plsc.subcore_barrier()
    pltpu.sync_copy(deg.at[pl.ds(stripe0, STRIPE)],
                    out.at[pl.ds(c * NPAD + stripe0, STRIPE)])


def _k1_body(deg_ref, x_ref, dis_ref, q_ref):
    d = deg_ref[0] + deg_ref[1] + 1.0
    r = lax.rsqrt(d)
    dis_ref[...] = r
    q_ref[...] = x_ref[...] * r


def _k3_body(agg_ref, q_ref, dis_ref, W1_ref, b1_ref, W2_ref, p2_ref, hw2_ref):
    a = agg_ref[0] + agg_ref[1] + q_ref[...]
    t1 = dis_ref[...] * a
    h1 = jnp.dot(t1, W1_ref[...], preferred_element_type=jnp.float32) + b1_ref[...]
    h1 = jnp.maximum(h1, 0.0)
    hw2 = jnp.dot(h1, W2_ref[...], preferred_element_type=jnp.float32)
    hw2_ref[...] = hw2
    p2 = dis_ref[...] * hw2
    p2_ref[0] = p2[:, :16]
    p2_ref[1] = p2[:, 16:]


def _k5_body(agg_ref, hw2_ref, dis_ref, batch_ref, b2_ref, Wf_ref, bf_ref,
             out_ref, acc_ref):
    i = pl.program_id(0)

    @pl.when(i == 0)
    def _():
        acc_ref[...] = jnp.zeros_like(acc_ref)

    aggcat = jnp.concatenate([agg_ref[0], agg_ref[1]], axis=1)
    z = dis_ref[...] * (aggcat + dis_ref[...] * hw2_ref[...]) + b2_ref[...]
    h2 = jnp.maximum(z, 0.0)
    oh = (batch_ref[...] == lax.broadcasted_iota(jnp.int32, (BLK, G), 1)).astype(jnp.float32)
    acc_ref[...] += lax.dot_general(
        oh, h2, (((0,), (0,)), ((), ())), preferred_element_type=jnp.float32)

    @pl.when(i == pl.num_programs(0) - 1)
    def _():
        gg = acc_ref[...]
        out_ref[...] = jax.nn.sigmoid(
            jnp.dot(gg, Wf_ref[...], preferred_element_type=jnp.float32)
            + bf_ref[...])


def kernel(x, edge_index, batch, W1, b1, W2, b2, Wf, bf):
    src = edge_index[0]
    dst = edge_index[1]
    srcp = jnp.concatenate([src, jnp.zeros((EPAD - E,), jnp.int32)])
    dstp = jnp.concatenate([dst, jnp.full((EPAD - E,), N, jnp.int32)])
    srcBp = srcp + NPAD
    xp = jnp.pad(x, ((0, NPAD - N), (0, 6)))
    batchp = jnp.pad(batch, (0, NPAD - N), constant_values=G)
    W1p = jnp.pad(W1, ((0, 6), (0, 0)))

    deg = _deg_kernel(dstp).reshape(2, NPAD, 1)

    nblk = NPAD // BLK
    dis, q = pl.pallas_call(
        _k1_body,
        grid=(nblk,),
        in_specs=[
            pl.BlockSpec((2, BLK, 1), lambda i: (0, i, 0)),
            pl.BlockSpec((BLK, 16), lambda i: (i, 0)),
        ],
        out_specs=[
            pl.BlockSpec((BLK, 1), lambda i: (i, 0)),
            pl.BlockSpec((BLK, 16), lambda i: (i, 0)),
        ],
        out_shape=[
            jax.ShapeDtypeStruct((NPAD, 1), jnp.float32),
            jax.ShapeDtypeStruct((NPAD, 16), jnp.float32),
        ],
    )(deg, xp)

    agg1 = _agg_l1(srcp, srcp, dstp, q)

    p2, hw2 = pl.pallas_call(
        _k3_body,
        grid=(nblk,),
        in_specs=[
            pl.BlockSpec((2, BLK, 16), lambda i: (0, i, 0)),
            pl.BlockSpec((BLK, 16), lambda i: (i, 0)),
            pl.BlockSpec((BLK, 1), lambda i: (i, 0)),
            pl.BlockSpec((16, 64), lambda i: (0, 0)),
            pl.BlockSpec((1, 64), lambda i: (0, 0)),
            pl.BlockSpec((64, 32), lambda i: (0, 0)),
        ],
        out_specs=[
            pl.BlockSpec((2, BLK, 16), lambda i: (0, i, 0)),
            pl.BlockSpec((BLK, 32), lambda i: (i, 0)),
        ],
        out_shape=[
            jax.ShapeDtypeStruct((2, NPAD, 16), jnp.float32),
            jax.ShapeDtypeStruct((NPAD, 32), jnp.float32),
        ],
    )(agg1, q, dis, W1p, b1.reshape(1, 64), W2)

    p2flat = p2.reshape(2 * NPAD, 16)
    agg2 = _agg_l2(srcp, srcBp, dstp, p2flat)

    out = pl.pallas_call(
        _k5_body,
        grid=(nblk,),
        in_specs=[
            pl.BlockSpec((2, BLK, 16), lambda i: (0, i, 0)),
            pl.BlockSpec((BLK, 32), lambda i: (i, 0)),
            pl.BlockSpec((BLK, 1), lambda i: (i, 0)),
            pl.BlockSpec((BLK, 1), lambda i: (i, 0)),
            pl.BlockSpec((1, 32), lambda i: (0, 0)),
            pl.BlockSpec((32, 1), lambda i: (0, 0)),
            pl.BlockSpec((1, 1), lambda i: (0, 0)),
        ],
        out_specs=pl.BlockSpec((G, 1), lambda i: (0, 0)),
        out_shape=jax.ShapeDtypeStruct((G, 1), jnp.float32),
        scratch_shapes=[pltpu.VMEM((G, 32), jnp.float32)],
    )(agg2, hw2, dis, batchp.reshape(NPAD, 1), b2.reshape(1, 32), Wf,
      bf.reshape(1, 1))

    return out
